# Initial kernel scaffold; baseline (speedup 1.0000x reference)
import jax
import jax.numpy as jnp
from jax.experimental import pallas as pl


def kernel(x, ldist, rdist, Wv, pf1, pf2):
    raise NotImplementedError("write your pallas kernel here")


# speedup vs baseline: 2.6942x; 2.6942x over previous
import functools

import jax
import jax.numpy as jnp
from jax import lax
from jax.experimental import pallas as pl
from jax.experimental.pallas import tpu as pltpu
from jax.experimental.pallas import tpu_sc as plsc

B, L = 4096, 50
N = B * L
D_W, D_F, D_OUT = 32, 16, 64
NC, NS = 2, 16
NW = NC * NS
GROUP = 128
G = 5
CHUNK = G * GROUP
GROUPS_PER_W = N // (NW * GROUP)
NCHUNK = GROUPS_PER_W // G

_mesh = plsc.VectorSubcoreMesh(
    core_axis_name="c", subcore_axis_name="s", num_cores=NC, num_subcores=NS
)


@functools.partial(
    pl.kernel,
    out_type=jax.ShapeDtypeStruct((N, D_OUT), jnp.float32),
    mesh=_mesh,
    compiler_params=pltpu.CompilerParams(use_tc_tiling_on_sc=False),
    scratch_types=[
        pltpu.VMEM((G, GROUP), jnp.int32),
        pltpu.VMEM((G, GROUP), jnp.int32),
        pltpu.VMEM((G, GROUP), jnp.int32),
        pltpu.VMEM((CHUNK, D_W), jnp.float32),
        pltpu.VMEM((CHUNK, D_F), jnp.float32),
        pltpu.VMEM((CHUNK, D_F), jnp.float32),
        pltpu.SemaphoreType.DMA,
    ],
)
def _emb_kernel(xi, li, ri, wv, pf1, pf2, out, xidx, lidx, ridx,
                wbuf, lbuf, rbuf, sem):
    wid = lax.axis_index("s") * NC + lax.axis_index("c")
    g_base = wid * GROUPS_PER_W

    def chunk_body(ci, carry):
        g0 = g_base + ci * G
        row0 = g0 * GROUP
        pltpu.sync_copy(xi.at[pl.ds(g0, G)], xidx)
        pltpu.sync_copy(li.at[pl.ds(g0, G)], lidx)
        pltpu.sync_copy(ri.at[pl.ds(g0, G)], ridx)
        copies = []
        for g in range(G):
            rows = pl.ds(g * GROUP, GROUP)
            copies.append(pltpu.async_copy(wv.at[xidx.at[g]], wbuf.at[rows], sem))
            copies.append(pltpu.async_copy(pf1.at[lidx.at[g]], lbuf.at[rows], sem))
            copies.append(pltpu.async_copy(pf2.at[ridx.at[g]], rbuf.at[rows], sem))
        for cp in copies:
            cp.wait()
        out_rows = pl.ds(row0, CHUNK)
        pltpu.sync_copy(wbuf, out.at[out_rows, pl.ds(0, D_W)])
        pltpu.sync_copy(lbuf, out.at[out_rows, pl.ds(D_W, D_F)])
        pltpu.sync_copy(rbuf, out.at[out_rows, pl.ds(D_W + D_F, D_F)])
        return carry

    lax.fori_loop(0, NCHUNK, chunk_body, 0)


def kernel(x, ldist, rdist, Wv, pf1, pf2):
    xi = x.reshape(-1).astype(jnp.int32).reshape(N // GROUP, GROUP)
    li = ldist.reshape(-1).astype(jnp.int32).reshape(N // GROUP, GROUP)
    ri = rdist.reshape(-1).astype(jnp.int32).reshape(N // GROUP, GROUP)
    out = _emb_kernel(xi, li, ri, Wv, pf1, pf2)
    return out.reshape(B, 1, L, D_OUT)


# speedup vs baseline: 2.6955x; 1.0005x over previous
import functools

import jax
import jax.numpy as jnp
from jax import lax
from jax.experimental import pallas as pl
from jax.experimental.pallas import tpu as pltpu
from jax.experimental.pallas import tpu_sc as plsc

B, L = 4096, 50
N = B * L
D_W, D_F, D_OUT = 32, 16, 64
NC, NS = 2, 16
NW = NC * NS
ROWS_PER_W = N // NW
CHUNK = 640
NCHUNK = ROWS_PER_W // CHUNK

_mesh = plsc.VectorSubcoreMesh(
    core_axis_name="c", subcore_axis_name="s", num_cores=NC, num_subcores=NS
)


@functools.partial(
    pl.kernel,
    out_type=jax.ShapeDtypeStruct((N, D_OUT), jnp.float32),
    mesh=_mesh,
    compiler_params=pltpu.CompilerParams(use_tc_tiling_on_sc=False),
    scratch_types=[
        pltpu.VMEM((CHUNK,), jnp.int32),
        pltpu.VMEM((CHUNK,), jnp.int32),
        pltpu.VMEM((CHUNK,), jnp.int32),
        pltpu.VMEM((CHUNK, D_W), jnp.float32),
        pltpu.VMEM((CHUNK, D_F), jnp.float32),
        pltpu.VMEM((CHUNK, D_F), jnp.float32),
        pltpu.SemaphoreType.DMA,
    ],
)
def _emb_kernel(xi, li, ri, wv, pf1, pf2, out, xidx, lidx, ridx,
                wbuf, lbuf, rbuf, sem):
    wid = lax.axis_index("s") * NC + lax.axis_index("c")
    base = wid * ROWS_PER_W

    def chunk_body(ci, carry):
        row0 = base + ci * CHUNK
        rows = pl.ds(row0, CHUNK)
        pltpu.sync_copy(xi.at[rows], xidx)
        pltpu.sync_copy(li.at[rows], lidx)
        pltpu.sync_copy(ri.at[rows], ridx)
        cw = pltpu.async_copy(wv.at[xidx], wbuf, sem)
        cl = pltpu.async_copy(pf1.at[lidx], lbuf, sem)
        cr = pltpu.async_copy(pf2.at[ridx], rbuf, sem)
        cw.wait()
        cl.wait()
        cr.wait()
        pltpu.sync_copy(wbuf, out.at[rows, pl.ds(0, D_W)])
        pltpu.sync_copy(lbuf, out.at[rows, pl.ds(D_W, D_F)])
        pltpu.sync_copy(rbuf, out.at[rows, pl.ds(D_W + D_F, D_F)])
        return carry

    lax.fori_loop(0, NCHUNK, chunk_body, 0)


def kernel(x, ldist, rdist, Wv, pf1, pf2):
    xi = x.reshape(-1).astype(jnp.int32)
    li = ldist.reshape(-1).astype(jnp.int32)
    ri = rdist.reshape(-1).astype(jnp.int32)
    out = _emb_kernel(xi, li, ri, Wv, pf1, pf2)
    return out.reshape(B, 1, L, D_OUT)


# speedup vs baseline: 2.7587x; 1.0234x over previous
import functools

import jax
import jax.numpy as jnp
from jax import lax
from jax.experimental import pallas as pl
from jax.experimental.pallas import tpu as pltpu
from jax.experimental.pallas import tpu_sc as plsc

B, L = 4096, 50
N = B * L
D_W, D_F, D_OUT = 32, 16, 64
NC, NS = 2, 16
NW = NC * NS
ROWS_PER_W = N // NW
CHUNK = 640
NCHUNK = ROWS_PER_W // CHUNK
NBUF = 2

_mesh = plsc.VectorSubcoreMesh(
    core_axis_name="c", subcore_axis_name="s", num_cores=NC, num_subcores=NS
)


@functools.partial(
    pl.kernel,
    out_type=jax.ShapeDtypeStruct((N, D_OUT), jnp.float32),
    mesh=_mesh,
    compiler_params=pltpu.CompilerParams(use_tc_tiling_on_sc=False),
    scratch_types=[
        pltpu.VMEM((ROWS_PER_W,), jnp.int32),
        pltpu.VMEM((ROWS_PER_W,), jnp.int32),
        pltpu.VMEM((ROWS_PER_W,), jnp.int32),
        [pltpu.VMEM((CHUNK, D_W), jnp.float32) for _ in range(NBUF)],
        [pltpu.VMEM((CHUNK, D_F), jnp.float32) for _ in range(NBUF)],
        [pltpu.VMEM((CHUNK, D_F), jnp.float32) for _ in range(NBUF)],
        [pltpu.SemaphoreType.DMA for _ in range(NBUF)],
        [pltpu.SemaphoreType.DMA for _ in range(NBUF)],
    ],
)
def _emb_kernel(xi, li, ri, wv, pf1, pf2, out, xidx, lidx, ridx,
                wbufs, lbufs, rbufs, gsems, wsems):
    wid = lax.axis_index("s") * NC + lax.axis_index("c")
    base = wid * ROWS_PER_W
    all_rows = pl.ds(base, ROWS_PER_W)
    pltpu.sync_copy(xi.at[all_rows], xidx)
    pltpu.sync_copy(li.at[all_rows], lidx)
    pltpu.sync_copy(ri.at[all_rows], ridx)

    def issue_gathers(ci, b):
        idx = pl.ds(ci * CHUNK, CHUNK)
        return [
            pltpu.async_copy(wv.at[xidx.at[idx]], wbufs[b], gsems[b]),
            pltpu.async_copy(pf1.at[lidx.at[idx]], lbufs[b], gsems[b]),
            pltpu.async_copy(pf2.at[ridx.at[idx]], rbufs[b], gsems[b]),
        ]

    def issue_writes(ci, b):
        rows = pl.ds(base + ci * CHUNK, CHUNK)
        return [
            pltpu.async_copy(wbufs[b], out.at[rows, pl.ds(0, D_W)], wsems[b]),
            pltpu.async_copy(lbufs[b], out.at[rows, pl.ds(D_W, D_F)], wsems[b]),
            pltpu.async_copy(rbufs[b], out.at[rows, pl.ds(D_W + D_F, D_F)], wsems[b]),
        ]

    gathers = {0: issue_gathers(0, 0)}
    writes = {}
    for ci in range(NCHUNK):
        b = ci % NBUF
        if ci + 1 < NCHUNK:
            if ci >= 1:
                for cp in writes[ci - 1]:
                    cp.wait()
            gathers[ci + 1] = issue_gathers(ci + 1, (ci + 1) % NBUF)
        for cp in gathers[ci]:
            cp.wait()
        writes[ci] = issue_writes(ci, b)
    for cp in writes[NCHUNK - 1]:
        cp.wait()
    for cp in writes[NCHUNK - 2]:
        cp.wait()


def kernel(x, ldist, rdist, Wv, pf1, pf2):
    xi = x.reshape(-1).astype(jnp.int32)
    li = ldist.reshape(-1).astype(jnp.int32)
    ri = rdist.reshape(-1).astype(jnp.int32)
    out = _emb_kernel(xi, li, ri, Wv, pf1, pf2)
    return out.reshape(B, 1, L, D_OUT)
